# Initial kernel scaffold; baseline (speedup 1.0000x reference)
import jax
import jax.numpy as jnp
from jax.experimental import pallas as pl


def kernel(latent_states, binary_mask, W, b):
    raise NotImplementedError("write your pallas kernel here")


# speedup vs baseline: 4.4433x; 4.4433x over previous
import functools

import jax
import jax.numpy as jnp
from jax import lax
from jax.experimental import pallas as pl
from jax.experimental.pallas import tpu as pltpu
from jax.experimental.pallas import tpu_sc as plsc

B, S, D = 16, 4096, 512
NP = 8
T = 32
LANES = 16


def _sc_pool_body(latent_hbm, mask_hbm, patches_hbm, mask_v, data_v, acc_v, off_v, cnt_s):
    wid = lax.axis_index("s") * 2 + lax.axis_index("c")

    @pl.when(wid < B)
    def _():
        row = wid
        l_iota = lax.iota(jnp.int32, 16)

        pltpu.sync_copy(mask_hbm.at[row], mask_v)

        off_v[...] = jnp.where(l_iota == 0, 0, S).astype(jnp.int32)

        cnt_s[0] = jnp.int32(0)

        @pl.loop(0, S // LANES)
        def scan_body(k):
            cnt = cnt_s[0]

            @pl.when(cnt < NP)
            def _():
                mv = mask_v[pl.ds(k * LANES, LANES)]
                cs = jnp.cumsum(mv)
                gid = cnt + cs
                sel = jnp.logical_and(mv > 0, gid <= NP)
                pos1 = k * LANES + l_iota + 1
                plsc.store_scatter(off_v, [jnp.minimum(gid, 15)], pos1, mask=sel)
                cnt_s[0] = cnt + jnp.sum(mv)

        off_reg = off_v[...]

        def off_at(j):
            return jnp.sum(jnp.where(l_iota == j, off_reg, 0))

        o8 = off_at(NP)

        @pl.loop(0, (NP * D) // LANES)
        def zero_body(i):
            acc_v[pl.ds(i * LANES, LANES)] = jnp.zeros((LANES,), jnp.float32)

        nchunks = (o8 + (T - 1)) // T

        @pl.loop(0, nchunks)
        def chunk_body(ci):
            c0 = ci * T
            pltpu.sync_copy(latent_hbm.at[row, pl.ds(c0 * D, T * D)], data_v)

            @pl.loop(0, NP)
            def patch_body(j):
                lo = jnp.maximum(off_at(j), c0)
                hi = jnp.minimum(off_at(j + 1), c0 + T)

                @pl.loop(lo, hi)
                def tok_body(t):
                    base = (t - c0) * D
                    for q in range(D // LANES):
                        sl = pl.ds(j * D + q * LANES, LANES)
                        acc_v[sl] = acc_v[sl] + data_v[pl.ds(base + q * LANES, LANES)]

        @pl.loop(0, NP)
        def scale_body(j):
            cnt = off_at(j + 1) - off_at(j)
            cntf = jnp.broadcast_to(cnt.astype(jnp.float32), (LANES,))
            inv = 1.0 / jnp.maximum(cntf, 1.0)
            for q in range(D // LANES):
                sl = pl.ds(j * D + q * LANES, LANES)
                acc_v[sl] = acc_v[sl] * inv

        pltpu.sync_copy(acc_v, patches_hbm.at[row])


_sc_pool = functools.partial(
    pl.kernel,
    mesh=plsc.VectorSubcoreMesh(core_axis_name="c", subcore_axis_name="s"),
    out_type=jax.ShapeDtypeStruct((B, NP * D), jnp.float32),
    scratch_types=[
        pltpu.VMEM((S,), jnp.int32),
        pltpu.VMEM((T * D,), jnp.float32),
        pltpu.VMEM((NP * D,), jnp.float32),
        pltpu.VMEM((16,), jnp.int32),
        pltpu.SMEM((1,), jnp.int32),
    ],
    compiler_params=pltpu.CompilerParams(needs_layout_passes=False),
)(_sc_pool_body)


def _proj_body(x_ref, w_ref, b_ref, o_ref):
    o_ref[...] = (
        lax.dot_general(
            x_ref[...],
            w_ref[...],
            dimension_numbers=(((1,), (1,)), ((), ())),
            preferred_element_type=jnp.float32,
        )
        + b_ref[...]
    )


_proj = pl.pallas_call(
    _proj_body,
    out_shape=jax.ShapeDtypeStruct((B * NP, D), jnp.float32),
)


def kernel(latent_states, binary_mask, W, b):
    latent2 = latent_states.reshape(B, S * D)
    mask2 = binary_mask.reshape(B, S).astype(jnp.int32)
    patches = _sc_pool(latent2, mask2)
    out = _proj(patches.reshape(B * NP, D), W, b.reshape(1, D))
    return out.reshape(B, NP, D)


# speedup vs baseline: 17.1724x; 3.8648x over previous
import functools

import jax
import jax.numpy as jnp
from jax import lax
from jax.experimental import pallas as pl
from jax.experimental.pallas import tpu as pltpu
from jax.experimental.pallas import tpu_sc as plsc

B, S, D = 16, 4096, 512
NP = 8
T = 32
LANES = 16
SM = S // 128


def _sc_pool_body(latent_hbm, mask_hbm, patches_hbm, mask_v, data_v, acc_v, off_v, cnt_s):
    wid = lax.axis_index("s") * 2 + lax.axis_index("c")

    @pl.when(wid < B)
    def _():
        row = wid
        l_iota = lax.iota(jnp.int32, 16)

        pltpu.sync_copy(mask_hbm.at[row], mask_v)

        off_v[...] = jnp.where(l_iota == 0, 0, S).astype(jnp.int32)

        cnt_s[0] = jnp.int32(0)

        @pl.loop(0, S // LANES)
        def scan_body(k):
            cnt = cnt_s[0]

            @pl.when(cnt < NP)
            def _():
                mv = mask_v[k // 8, pl.ds((k % 8) * LANES, LANES)]
                cs = jnp.cumsum(mv)
                gid = cnt + cs
                sel = jnp.logical_and(mv > 0, gid <= NP)
                pos1 = k * LANES + l_iota + 1
                plsc.store_scatter(off_v, [jnp.minimum(gid, 15)], pos1, mask=sel)
                cnt_s[0] = cnt + jnp.sum(mv)

        off_reg = off_v[...]

        def off_at(j):
            return jnp.sum(jnp.where(l_iota == j, off_reg, 0))

        o8 = off_at(NP)

        @pl.loop(0, NP)
        def zero_body(j):
            for q in range(D // LANES):
                acc_v[j, pl.ds(q * LANES, LANES)] = jnp.zeros((LANES,), jnp.float32)

        nchunks = (o8 + (T - 1)) // T

        @pl.loop(0, nchunks)
        def chunk_body(ci):
            c0 = ci * T
            pltpu.sync_copy(latent_hbm.at[row, pl.ds(c0, T), :], data_v)

            @pl.loop(0, NP)
            def patch_body(j):
                lo = jnp.maximum(off_at(j), c0)
                hi = jnp.minimum(off_at(j + 1), c0 + T)

                @pl.loop(lo, hi)
                def tok_body(t):
                    for q in range(D // LANES):
                        sl = pl.ds(q * LANES, LANES)
                        acc_v[j, sl] = acc_v[j, sl] + data_v[t - c0, sl]

        @pl.loop(0, NP)
        def scale_body(j):
            cnt = off_at(j + 1) - off_at(j)
            cntf = jnp.broadcast_to(cnt.astype(jnp.float32), (LANES,))
            inv = 1.0 / jnp.maximum(cntf, 1.0)
            for q in range(D // LANES):
                sl = pl.ds(q * LANES, LANES)
                acc_v[j, sl] = acc_v[j, sl] * inv

        pltpu.sync_copy(acc_v, patches_hbm.at[row])


_sc_pool = functools.partial(
    pl.kernel,
    mesh=plsc.VectorSubcoreMesh(core_axis_name="c", subcore_axis_name="s"),
    out_type=jax.ShapeDtypeStruct((B, NP, D), jnp.float32),
    scratch_types=[
        pltpu.VMEM((SM, 128), jnp.int32),
        pltpu.VMEM((T, D), jnp.float32),
        pltpu.VMEM((NP, D), jnp.float32),
        pltpu.VMEM((16,), jnp.int32),
        pltpu.SMEM((1,), jnp.int32),
    ],
    compiler_params=pltpu.CompilerParams(
        needs_layout_passes=False, use_tc_tiling_on_sc=True
    ),
)(_sc_pool_body)


def _proj_body(x_ref, w_ref, b_ref, o_ref):
    o_ref[...] = (
        lax.dot_general(
            x_ref[...],
            w_ref[...],
            dimension_numbers=(((1,), (1,)), ((), ())),
            preferred_element_type=jnp.float32,
        )
        + b_ref[...]
    )


_proj = pl.pallas_call(
    _proj_body,
    out_shape=jax.ShapeDtypeStruct((B * NP, D), jnp.float32),
)


def kernel(latent_states, binary_mask, W, b):
    mask2 = binary_mask.reshape(B, SM, 128).astype(jnp.int32)
    patches = _sc_pool(latent_states, mask2)
    out = _proj(patches.reshape(B * NP, D), W, b.reshape(1, D))
    return out.reshape(B, NP, D)
